# baseline (speedup 1.0000x reference)
import jax
import jax.numpy as jnp
from jax import lax
from jax.experimental import pallas as pl
from jax.experimental.pallas import tpu as pltpu
from jax.experimental.pallas import tpu_sc as plsc

_B = 8
_N = 5000
_LANES = 16
_HALF = 2560
_STRIDE = 2440
_SBUF = 2688
_CHUNKS = 160
_NUM_DET = 3
_UNROLL = 8


def _splat(x):
    return jnp.full((_LANES,), x)


def _nms_body(arr_h, out_h,
              s_v, x1_v, y1_v, x2_v, y2_v, c_v, e_v, p_v, ob_v, o2_v,
              t_v, ti_v, slab, sem_s, sem_bc):
    s = lax.axis_index("s")
    half = jnp.where(s >= _B, 1, 0)
    b = s - _B * half
    start = half * _STRIDE
    partner = jnp.where(s >= _B, s - _B, s + _B)

    cp_s = pltpu.async_copy(arr_h.at[s, 0], s_v.at[pl.ds(0, _HALF)], sem_s)
    rest = [pltpu.async_copy(arr_h.at[s, j], dst.at[pl.ds(0, _HALF)], sem_bc)
            for j, dst in ((1, x1_v), (2, y1_v), (3, x2_v), (4, y2_v),
                           (5, c_v))]

    lanes = lax.iota(jnp.int32, _LANES)
    neg = jnp.float32(-jnp.inf)

    cp_s.wait()

    def select_pass(prev, store):
        def chunk(i, carry):
            bv, bi = carry
            base = i * _LANES
            sl = pl.ds(base, _LANES)
            idx = base + lanes
            sc = s_v[sl]
            if prev is not None:
                wx1, wy1, wx2, wy2, wcls, warea = prev
                bx1 = x1_v[sl]
                by1 = y1_v[sl]
                bx2 = x2_v[sl]
                by2 = y2_v[sl]
                cc = c_v[sl]
                ix1 = jnp.maximum(wx1, bx1)
                iy1 = jnp.maximum(wy1, by1)
                ix2 = jnp.minimum(wx2, bx2)
                iy2 = jnp.minimum(wy2, by2)
                inter = (jnp.maximum(ix2 - ix1, 0.0)
                         * jnp.maximum(iy2 - iy1, 0.0))
                area_b = (bx2 - bx1) * (by2 - by1)
                kill = ((inter > 0.5 * (warea + area_b - inter))
                        & (cc == wcls))
                sc = jnp.where(kill, neg, sc)
                if store:
                    s_v[sl] = sc
            take = sc > bv
            return jnp.where(take, sc, bv), jnp.where(take, idx, bi)

        bv0 = jnp.full((_LANES,), neg, jnp.float32)
        bi0 = jnp.zeros((_LANES,), jnp.int32)
        bv, bi = lax.fori_loop(0, _CHUNKS, chunk, (bv0, bi0),
                               unroll=_UNROLL)
        v, i = bv, bi
        for d in (8, 4, 2, 1):
            t_v[pl.ds(0, _LANES)] = v
            t_v[pl.ds(_LANES, _LANES)] = v
            ti_v[pl.ds(0, _LANES)] = i
            ti_v[pl.ds(_LANES, _LANES)] = i
            v2 = t_v[pl.ds(d, _LANES)]
            i2 = ti_v[pl.ds(d, _LANES)]
            better = (v2 > v) | ((v2 == v) & (i2 < i))
            v = jnp.where(better, v2, v)
            i = jnp.where(better, i2, i)
        return v, i

    def exchange(r, mv, mi):
        gi_v = (mi + start).astype(jnp.float32)
        mi_s = mi[0]
        sl = pl.ds(mi_s, _LANES)
        wx1 = _splat(x1_v[sl][0])
        wy1 = _splat(y1_v[sl][0])
        wx2 = _splat(x2_v[sl][0])
        wy2 = _splat(y2_v[sl][0])
        wcl = _splat(c_v[sl][0])
        wa = (wx2 - wx1) * (wy2 - wy1)
        t = mv
        t = jnp.where(lanes == 1, gi_v, t)
        t = jnp.where(lanes == 2, wx1, t)
        t = jnp.where(lanes == 3, wy1, t)
        t = jnp.where(lanes == 4, wx2, t)
        t = jnp.where(lanes == 5, wy2, t)
        t = jnp.where(lanes == 6, wcl, t)
        t = jnp.where(lanes == 7, wa, t)
        e_v[...] = t
        pltpu.sync_copy(e_v, slab.at[pl.ds((r * 16 + s) * _LANES, _LANES)])
        plsc.subcore_barrier()
        pltpu.sync_copy(slab.at[pl.ds((r * 16 + partner) * _LANES, _LANES)],
                        p_v)
        pv = p_v[...]
        pbetter = (pv[0] > mv[0]) | ((pv[0] == mv[0]) & (pv[1] < gi_v[0]))
        return jnp.where(pbetter, pv, t)

    def winner_splats(wt):
        return (_splat(wt[2]), _splat(wt[3]), _splat(wt[4]), _splat(wt[5]),
                _splat(wt[6]), _splat(wt[7]))

    mv0, mi0 = select_pass(None, False)
    wt0 = exchange(0, mv0, mi0)
    g0 = wt0[1].astype(jnp.int32)

    for cp in rest:
        cp.wait()

    mv1, mi1 = select_pass(winner_splats(wt0), True)
    wt1 = exchange(1, mv1, mi1)
    g1 = wt1[1].astype(jnp.int32)

    mv2, mi2 = select_pass(winner_splats(wt1), False)
    wt2 = exchange(2, mv2, mi2)
    g2 = wt2[1].astype(jnp.int32)

    ov = jnp.where(lanes == 0, g0,
         jnp.where(lanes == 1, g1,
         jnp.where(lanes == 2, g2, 0)))
    e_v[...] = ov.astype(jnp.float32)
    pltpu.sync_copy(e_v, slab.at[pl.ds((3 * 16 + s) * _LANES, _LANES)])
    plsc.subcore_barrier()

    @pl.when(s == 0)
    def _():
        pltpu.sync_copy(slab.at[pl.ds(3 * 16 * _LANES, _B * _LANES)], ob_v)
        r = [ob_v[pl.ds(i * _LANES, _LANES)] for i in range(_B)]
        vals = {}
        for img in range(_B):
            for k in range(_NUM_DET):
                vals[img * _NUM_DET + k] = r[img][k]
        f0 = _splat(vals[0])
        for j in range(1, 16):
            f0 = jnp.where(lanes == j, _splat(vals[j]), f0)
        f1 = _splat(vals[8])
        for j in range(1, 16):
            f1 = jnp.where(lanes == j, _splat(vals[8 + j]), f1)
        o2_v[pl.ds(0, _LANES)] = f0.astype(jnp.int32)
        o2_v[pl.ds(8, _LANES)] = f1.astype(jnp.int32)
        pltpu.sync_copy(o2_v, out_h)


def kernel(scores, boxes, classes):
    planes = jnp.stack(
        [scores,
         boxes[..., 0], boxes[..., 1], boxes[..., 2], boxes[..., 3],
         classes.astype(jnp.float32)],
        axis=1)
    arr = jnp.concatenate(
        [planes[:, :, :_HALF], planes[:, :, _STRIDE:]], axis=0)

    call = pl.kernel(
        _nms_body,
        out_type=jax.ShapeDtypeStruct((_B * _NUM_DET,), jnp.int32),
        mesh=plsc.VectorSubcoreMesh(core_axis_name="c", subcore_axis_name="s",
                                    num_cores=1),
        scratch_types=[
            pltpu.VMEM((_SBUF,), jnp.float32),
            pltpu.VMEM((_SBUF,), jnp.float32),
            pltpu.VMEM((_SBUF,), jnp.float32),
            pltpu.VMEM((_SBUF,), jnp.float32),
            pltpu.VMEM((_SBUF,), jnp.float32),
            pltpu.VMEM((_SBUF,), jnp.float32),
            pltpu.VMEM((_LANES,), jnp.float32),
            pltpu.VMEM((_LANES,), jnp.float32),
            pltpu.VMEM((_B * _LANES,), jnp.float32),
            pltpu.VMEM((_B * _NUM_DET,), jnp.int32),
            pltpu.VMEM((2 * _LANES,), jnp.float32),
            pltpu.VMEM((2 * _LANES,), jnp.int32),
            pltpu.VMEM_SHARED((4 * 16 * _LANES,), jnp.float32),
            pltpu.SemaphoreType.DMA,
            pltpu.SemaphoreType.DMA,
        ],
    )
    out = call(arr)
    return out.reshape(_B, _NUM_DET)


# speedup vs baseline: 1.0654x; 1.0654x over previous
import jax
import jax.numpy as jnp
from jax import lax
from jax.experimental import pallas as pl
from jax.experimental.pallas import tpu as pltpu
from jax.experimental.pallas import tpu_sc as plsc

_B = 8
_N = 5000
_LANES = 16
_HALF = 2560
_STRIDE = 2440
_SBUF = 2688
_CHUNKS = 160
_NUM_DET = 3
_UNROLL = 4


def _splat(x):
    return jnp.full((_LANES,), x)


def _nms_body(arr_h, out_h,
              s_v, x1_v, y1_v, x2_v, y2_v, c_v, e_v, p_v, ob_v, o2_v,
              t_v, ti_v, slab, sem_s, sem_bc):
    s = lax.axis_index("s")
    half = jnp.where(s >= _B, 1, 0)
    b = s - _B * half
    start = half * _STRIDE
    partner = jnp.where(s >= _B, s - _B, s + _B)

    cp_s = pltpu.async_copy(arr_h.at[s, 0], s_v.at[pl.ds(0, _HALF)], sem_s)
    rest = [pltpu.async_copy(arr_h.at[s, j], dst.at[pl.ds(0, _HALF)], sem_bc)
            for j, dst in ((1, x1_v), (2, y1_v), (3, x2_v), (4, y2_v),
                           (5, c_v))]

    lanes = lax.iota(jnp.int32, _LANES)
    neg = jnp.float32(-jnp.inf)

    cp_s.wait()

    def select_pass(prev, store):
        def chunk(i, carry):
            bv, bi = carry
            base = i * _LANES
            sl = pl.ds(base, _LANES)
            idx = base + lanes
            sc = s_v[sl]
            if prev is not None:
                wx1, wy1, wx2, wy2, wcls, warea = prev
                bx1 = x1_v[sl]
                by1 = y1_v[sl]
                bx2 = x2_v[sl]
                by2 = y2_v[sl]
                cc = c_v[sl]
                ix1 = jnp.maximum(wx1, bx1)
                iy1 = jnp.maximum(wy1, by1)
                ix2 = jnp.minimum(wx2, bx2)
                iy2 = jnp.minimum(wy2, by2)
                inter = (jnp.maximum(ix2 - ix1, 0.0)
                         * jnp.maximum(iy2 - iy1, 0.0))
                area_b = (bx2 - bx1) * (by2 - by1)
                kill = ((inter > 0.5 * (warea + area_b - inter))
                        & (cc == wcls))
                sc = jnp.where(kill, neg, sc)
                if store:
                    s_v[sl] = sc
            take = sc > bv
            return jnp.where(take, sc, bv), jnp.where(take, idx, bi)

        bv0 = jnp.full((_LANES,), neg, jnp.float32)
        bi0 = jnp.zeros((_LANES,), jnp.int32)
        bv, bi = lax.fori_loop(0, _CHUNKS, chunk, (bv0, bi0),
                               unroll=_UNROLL)
        v, i = bv, bi
        for d in (8, 4, 2, 1):
            t_v[pl.ds(0, _LANES)] = v
            t_v[pl.ds(_LANES, _LANES)] = v
            ti_v[pl.ds(0, _LANES)] = i
            ti_v[pl.ds(_LANES, _LANES)] = i
            v2 = t_v[pl.ds(d, _LANES)]
            i2 = ti_v[pl.ds(d, _LANES)]
            better = (v2 > v) | ((v2 == v) & (i2 < i))
            v = jnp.where(better, v2, v)
            i = jnp.where(better, i2, i)
        return v, i

    def exchange(r, mv, mi):
        gi_v = (mi + start).astype(jnp.float32)
        mi_s = mi[0]
        sl = pl.ds(mi_s, _LANES)
        wx1 = _splat(x1_v[sl][0])
        wy1 = _splat(y1_v[sl][0])
        wx2 = _splat(x2_v[sl][0])
        wy2 = _splat(y2_v[sl][0])
        wcl = _splat(c_v[sl][0])
        wa = (wx2 - wx1) * (wy2 - wy1)
        t = mv
        t = jnp.where(lanes == 1, gi_v, t)
        t = jnp.where(lanes == 2, wx1, t)
        t = jnp.where(lanes == 3, wy1, t)
        t = jnp.where(lanes == 4, wx2, t)
        t = jnp.where(lanes == 5, wy2, t)
        t = jnp.where(lanes == 6, wcl, t)
        t = jnp.where(lanes == 7, wa, t)
        e_v[...] = t
        pltpu.sync_copy(e_v, slab.at[pl.ds((r * 16 + s) * _LANES, _LANES)])
        plsc.subcore_barrier()
        pltpu.sync_copy(slab.at[pl.ds((r * 16 + partner) * _LANES, _LANES)],
                        p_v)
        pv = p_v[...]
        pbetter = (pv[0] > mv[0]) | ((pv[0] == mv[0]) & (pv[1] < gi_v[0]))
        return jnp.where(pbetter, pv, t)

    def winner_splats(wt):
        return (_splat(wt[2]), _splat(wt[3]), _splat(wt[4]), _splat(wt[5]),
                _splat(wt[6]), _splat(wt[7]))

    mv0, mi0 = select_pass(None, False)
    wt0 = exchange(0, mv0, mi0)
    g0 = wt0[1].astype(jnp.int32)

    for cp in rest:
        cp.wait()

    mv1, mi1 = select_pass(winner_splats(wt0), True)
    wt1 = exchange(1, mv1, mi1)
    g1 = wt1[1].astype(jnp.int32)

    mv2, mi2 = select_pass(winner_splats(wt1), False)
    wt2 = exchange(2, mv2, mi2)
    g2 = wt2[1].astype(jnp.int32)

    ov = jnp.where(lanes == 0, g0,
         jnp.where(lanes == 1, g1,
         jnp.where(lanes == 2, g2, 0)))
    e_v[...] = ov.astype(jnp.float32)
    pltpu.sync_copy(e_v, slab.at[pl.ds((3 * 16 + s) * _LANES, _LANES)])
    plsc.subcore_barrier()

    @pl.when(s == 0)
    def _():
        pltpu.sync_copy(slab.at[pl.ds(3 * 16 * _LANES, _B * _LANES)], ob_v)
        r = [ob_v[pl.ds(i * _LANES, _LANES)] for i in range(_B)]
        vals = {}
        for img in range(_B):
            for k in range(_NUM_DET):
                vals[img * _NUM_DET + k] = r[img][k]
        f0 = _splat(vals[0])
        for j in range(1, 16):
            f0 = jnp.where(lanes == j, _splat(vals[j]), f0)
        f1 = _splat(vals[8])
        for j in range(1, 16):
            f1 = jnp.where(lanes == j, _splat(vals[8 + j]), f1)
        o2_v[pl.ds(0, _LANES)] = f0.astype(jnp.int32)
        o2_v[pl.ds(8, _LANES)] = f1.astype(jnp.int32)
        pltpu.sync_copy(o2_v, out_h)


def kernel(scores, boxes, classes):
    planes = jnp.stack(
        [scores,
         boxes[..., 0], boxes[..., 1], boxes[..., 2], boxes[..., 3],
         classes.astype(jnp.float32)],
        axis=1)
    arr = jnp.concatenate(
        [planes[:, :, :_HALF], planes[:, :, _STRIDE:]], axis=0)

    call = pl.kernel(
        _nms_body,
        out_type=jax.ShapeDtypeStruct((_B * _NUM_DET,), jnp.int32),
        mesh=plsc.VectorSubcoreMesh(core_axis_name="c", subcore_axis_name="s",
                                    num_cores=1),
        scratch_types=[
            pltpu.VMEM((_SBUF,), jnp.float32),
            pltpu.VMEM((_SBUF,), jnp.float32),
            pltpu.VMEM((_SBUF,), jnp.float32),
            pltpu.VMEM((_SBUF,), jnp.float32),
            pltpu.VMEM((_SBUF,), jnp.float32),
            pltpu.VMEM((_SBUF,), jnp.float32),
            pltpu.VMEM((_LANES,), jnp.float32),
            pltpu.VMEM((_LANES,), jnp.float32),
            pltpu.VMEM((_B * _LANES,), jnp.float32),
            pltpu.VMEM((_B * _NUM_DET,), jnp.int32),
            pltpu.VMEM((2 * _LANES,), jnp.float32),
            pltpu.VMEM((2 * _LANES,), jnp.int32),
            pltpu.VMEM_SHARED((4 * 16 * _LANES,), jnp.float32),
            pltpu.SemaphoreType.DMA,
            pltpu.SemaphoreType.DMA,
        ],
    )
    out = call(arr)
    return out.reshape(_B, _NUM_DET)
